# baseline (speedup 1.0000x reference)
import functools

import jax
import jax.numpy as jnp
from jax.experimental import pallas as pl

B, S, H = 1, 2048, 1024
E, K, M = 8, 2, 2048
T = B * S
ALPHA = 1.702
LIMIT = 7.0

MT = 512
NMT = M // MT


def _router_body(x_ref, wr_ref, br_ref, scores_ref):
    x = x_ref[...]
    logits = jnp.dot(x, wr_ref[...], preferred_element_type=jnp.float32)
    logits = logits + br_ref[...][None, :]
    iota = jax.lax.broadcasted_iota(jnp.int32, (T, E), 1)
    neg_inf = jnp.float32(-jnp.inf)

    m1 = jnp.max(logits, axis=1, keepdims=True)
    i1 = jnp.min(jnp.where(logits == m1, iota, E), axis=1, keepdims=True)
    masked = jnp.where(iota == i1, neg_inf, logits)
    m2 = jnp.max(masked, axis=1, keepdims=True)
    i2 = jnp.min(jnp.where(masked == m2, iota, E), axis=1, keepdims=True)

    e2 = jnp.exp(m2 - m1)
    denom = 1.0 + e2
    w1 = 1.0 / denom
    w2 = e2 / denom
    scores_ref[...] = jnp.where(iota == i1, w1, 0.0) + jnp.where(iota == i2, w2, 0.0)


def _expert_body(x_ref, wg_ref, wu_ref, w2_ref, bg_ref, bu_ref, bd_ref,
                 s_ref, out_ref):
    e = pl.program_id(0)
    m = pl.program_id(1)

    x = x_ref[...].astype(jnp.bfloat16)
    wg = wg_ref[0].astype(jnp.bfloat16)
    wu = wu_ref[0].astype(jnp.bfloat16)
    gate = jnp.dot(x, wg, preferred_element_type=jnp.float32)
    gate = gate + bg_ref[0]
    up = jnp.dot(x, wu, preferred_element_type=jnp.float32)
    up = up + bu_ref[0]

    gate = jnp.clip(gate, -1e9, LIMIT)
    up = jnp.clip(up, -LIMIT, LIMIT)
    glu = gate * jax.nn.sigmoid(gate * ALPHA)
    act = (up + 1.0) * glu

    onehot = (jax.lax.broadcasted_iota(jnp.int32, (E, 1), 0) == e
              ).astype(jnp.float32)
    s_col = jnp.dot(s_ref[...], onehot, preferred_element_type=jnp.float32)

    partial = jnp.dot((act * s_col).astype(jnp.bfloat16),
                      w2_ref[0].astype(jnp.bfloat16),
                      preferred_element_type=jnp.float32)

    @pl.when(m == 0)
    def _():
        partial_b = partial + s_col * bd_ref[0]

        @pl.when(e == 0)
        def _():
            out_ref[...] = partial_b

        @pl.when(e != 0)
        def _():
            out_ref[...] += partial_b

    @pl.when(m != 0)
    def _():
        out_ref[...] += partial


@jax.jit
def kernel(hidden_states, router_kernel, router_bias, gate_up_proj,
           gate_up_proj_bias, down_proj, down_proj_bias):
    flat = hidden_states.reshape(T, H)

    scores = pl.pallas_call(
        _router_body,
        out_shape=jax.ShapeDtypeStruct((T, E), jnp.float32),
    )(flat, router_kernel, router_bias)

    gu = gate_up_proj.reshape(E, H, M, 2)
    wg = gu[..., 0]
    wu = gu[..., 1]
    gub = gate_up_proj_bias.reshape(E, M, 2)
    bg = gub[..., 0].reshape(E, 1, M)
    bu = gub[..., 1].reshape(E, 1, M)
    bd = down_proj_bias.reshape(E, 1, H)

    out = pl.pallas_call(
        _expert_body,
        grid=(E, NMT),
        in_specs=[
            pl.BlockSpec((T, H), lambda e, m: (0, 0)),
            pl.BlockSpec((1, H, MT), lambda e, m: (e, 0, m)),
            pl.BlockSpec((1, H, MT), lambda e, m: (e, 0, m)),
            pl.BlockSpec((1, MT, H), lambda e, m: (e, m, 0)),
            pl.BlockSpec((1, 1, MT), lambda e, m: (e, 0, m)),
            pl.BlockSpec((1, 1, MT), lambda e, m: (e, 0, m)),
            pl.BlockSpec((1, 1, H), lambda e, m: (e, 0, 0)),
            pl.BlockSpec((T, E), lambda e, m: (0, 0)),
        ],
        out_specs=pl.BlockSpec((T, H), lambda e, m: (0, 0)),
        out_shape=jax.ShapeDtypeStruct((T, H), jnp.float32),
    )(flat, wg, wu, down_proj, bg, bu, bd, scores)

    return out.reshape(B, S, H), scores


# speedup vs baseline: 1.1027x; 1.1027x over previous
import functools

import jax
import jax.numpy as jnp
from jax.experimental import pallas as pl
from jax.experimental.pallas import tpu as pltpu

B, S, H = 1, 2048, 1024
E, K, M = 8, 2, 2048
T = B * S
TK = T * K
ALPHA = 1.702
LIMIT = 7.0

RB = 128
PAD_ROWS = TK + E * RB
NT = PAD_ROWS // RB


def _router_body(x_ref, wr_ref, br_ref, scores_ref, eids_ref, ws_ref):
    x = x_ref[...]
    logits = jnp.dot(x, wr_ref[...], preferred_element_type=jnp.float32)
    logits = logits + br_ref[...][None, :]
    iota = jax.lax.broadcasted_iota(jnp.int32, (T, E), 1)
    neg_inf = jnp.float32(-jnp.inf)

    m1 = jnp.max(logits, axis=1, keepdims=True)
    i1 = jnp.min(jnp.where(logits == m1, iota, E), axis=1, keepdims=True)
    masked = jnp.where(iota == i1, neg_inf, logits)
    m2 = jnp.max(masked, axis=1, keepdims=True)
    i2 = jnp.min(jnp.where(masked == m2, iota, E), axis=1, keepdims=True)

    e2 = jnp.exp(m2 - m1)
    denom = 1.0 + e2
    w1 = 1.0 / denom
    w2 = e2 / denom
    scores_ref[...] = jnp.where(iota == i1, w1, 0.0) + jnp.where(iota == i2, w2, 0.0)
    eids_ref[...] = jnp.concatenate([i1, i2], axis=1)
    ws_ref[...] = jnp.concatenate([w1, w2], axis=1)


def _moe_body(te_ref, x_ref, wg_ref, wu_ref, w2_ref, bg_ref, bu_ref, bd_ref,
              out_ref):
    x = x_ref[...]
    gate = jnp.dot(x, wg_ref[0], preferred_element_type=jnp.float32)
    gate = gate + bg_ref[0]
    up = jnp.dot(x, wu_ref[0], preferred_element_type=jnp.float32)
    up = up + bu_ref[0]

    gate = jnp.clip(gate, -1e9, LIMIT)
    up = jnp.clip(up, -LIMIT, LIMIT)
    glu = gate * jax.nn.sigmoid(gate * ALPHA)
    act = (up + 1.0) * glu

    out_ref[...] = jnp.dot(act, w2_ref[0],
                           preferred_element_type=jnp.float32) + bd_ref[0]


@jax.jit
def kernel(hidden_states, router_kernel, router_bias, gate_up_proj,
           gate_up_proj_bias, down_proj, down_proj_bias):
    flat = hidden_states.reshape(T, H)

    scores, eids, ws = pl.pallas_call(
        _router_body,
        out_shape=(
            jax.ShapeDtypeStruct((T, E), jnp.float32),
            jax.ShapeDtypeStruct((T, K), jnp.int32),
            jax.ShapeDtypeStruct((T, K), jnp.float32),
        ),
    )(flat, router_kernel, router_bias)

    flat_e = eids.reshape(TK)
    perm = jnp.argsort(flat_e, stable=True)
    tok_sorted = perm // K
    g_sorted = flat_e[perm]
    counts = jnp.bincount(flat_e, length=E)
    padded = ((counts + RB - 1) // RB) * RB
    cstarts = jnp.concatenate([jnp.zeros((1,), jnp.int32),
                               jnp.cumsum(padded)[:-1].astype(jnp.int32)])
    ustarts = jnp.concatenate([jnp.zeros((1,), jnp.int32),
                               jnp.cumsum(counts)[:-1].astype(jnp.int32)])
    j = jnp.arange(TK, dtype=jnp.int32)
    pos = cstarts[g_sorted] + j - ustarts[g_sorted]
    rows_padded = jnp.zeros((PAD_ROWS,), jnp.int32).at[pos].set(tok_sorted)
    inv = jnp.zeros((TK,), jnp.int32).at[perm].set(pos).reshape(T, K)
    cends = cstarts + padded
    tile_expert = jnp.minimum(
        jnp.searchsorted(cends, jnp.arange(NT, dtype=jnp.int32) * RB,
                         side="right"),
        E - 1).astype(jnp.int32)

    x_sorted = flat[rows_padded]

    gu = gate_up_proj.reshape(E, H, M, 2)
    wg = gu[..., 0]
    wu = gu[..., 1]
    gub = gate_up_proj_bias.reshape(E, M, 2)
    bg = gub[..., 0].reshape(E, 1, M)
    bu = gub[..., 1].reshape(E, 1, M)
    bd = down_proj_bias.reshape(E, 1, H)

    out_sorted = pl.pallas_call(
        _moe_body,
        grid_spec=pltpu.PrefetchScalarGridSpec(
            num_scalar_prefetch=1,
            grid=(NT,),
            in_specs=[
                pl.BlockSpec((RB, H), lambda i, te: (i, 0)),
                pl.BlockSpec((1, H, M), lambda i, te: (te[i], 0, 0)),
                pl.BlockSpec((1, H, M), lambda i, te: (te[i], 0, 0)),
                pl.BlockSpec((1, M, H), lambda i, te: (te[i], 0, 0)),
                pl.BlockSpec((1, 1, M), lambda i, te: (te[i], 0, 0)),
                pl.BlockSpec((1, 1, M), lambda i, te: (te[i], 0, 0)),
                pl.BlockSpec((1, 1, H), lambda i, te: (te[i], 0, 0)),
            ],
            out_specs=pl.BlockSpec((RB, H), lambda i, te: (i, 0)),
        ),
        out_shape=jax.ShapeDtypeStruct((PAD_ROWS, H), jnp.float32),
    )(tile_expert, x_sorted, wg, wu, down_proj, bg, bu, bd)

    final = (ws[:, 0:1] * out_sorted[inv[:, 0]]
             + ws[:, 1:2] * out_sorted[inv[:, 1]])
    return final.reshape(B, S, H), scores


# speedup vs baseline: 1.1046x; 1.0017x over previous
import functools

import jax
import jax.numpy as jnp
from jax.experimental import pallas as pl
from jax.experimental.pallas import tpu as pltpu

B, S, H = 1, 2048, 1024
E, K, M = 8, 2, 2048
T = B * S
TK = T * K
ALPHA = 1.702
LIMIT = 7.0

RB = 128
PAD_ROWS = TK + E * RB
NT = PAD_ROWS // RB


def _router_body(x_ref, wr_ref, br_ref, scores_ref, eids_ref, ws_ref):
    x = x_ref[...]
    logits = jnp.dot(x, wr_ref[...], preferred_element_type=jnp.float32)
    logits = logits + br_ref[...][None, :]
    iota = jax.lax.broadcasted_iota(jnp.int32, (T, E), 1)
    neg_inf = jnp.float32(-jnp.inf)

    m1 = jnp.max(logits, axis=1, keepdims=True)
    i1 = jnp.min(jnp.where(logits == m1, iota, E), axis=1, keepdims=True)
    masked = jnp.where(iota == i1, neg_inf, logits)
    m2 = jnp.max(masked, axis=1, keepdims=True)
    i2 = jnp.min(jnp.where(masked == m2, iota, E), axis=1, keepdims=True)

    e2 = jnp.exp(m2 - m1)
    denom = 1.0 + e2
    w1 = 1.0 / denom
    w2 = e2 / denom
    scores_ref[...] = jnp.where(iota == i1, w1, 0.0) + jnp.where(iota == i2, w2, 0.0)
    eids_ref[...] = jnp.concatenate([i1, i2], axis=1)
    ws_ref[...] = jnp.concatenate([w1, w2], axis=1)


def _moe_body(te_ref, x_ref, wg_ref, wu_ref, w2_ref, bg_ref, bu_ref, bd_ref,
              out_ref):
    x = x_ref[...].astype(jnp.bfloat16)
    gate = jnp.dot(x, wg_ref[0], preferred_element_type=jnp.float32)
    up = jnp.dot(x, wu_ref[0], preferred_element_type=jnp.float32)
    gate = gate + bg_ref[0]
    up = up + bu_ref[0]

    gate = jnp.clip(gate, -1e9, LIMIT)
    up = jnp.clip(up, -LIMIT, LIMIT)
    glu = gate * jax.nn.sigmoid(gate * ALPHA)
    act = (up + 1.0) * glu

    out_ref[...] = jnp.dot(act.astype(jnp.bfloat16), w2_ref[0],
                           preferred_element_type=jnp.float32) + bd_ref[0]


@jax.jit
def kernel(hidden_states, router_kernel, router_bias, gate_up_proj,
           gate_up_proj_bias, down_proj, down_proj_bias):
    flat = hidden_states.reshape(T, H)

    scores, eids, ws = pl.pallas_call(
        _router_body,
        out_shape=(
            jax.ShapeDtypeStruct((T, E), jnp.float32),
            jax.ShapeDtypeStruct((T, K), jnp.int32),
            jax.ShapeDtypeStruct((T, K), jnp.float32),
        ),
    )(flat, router_kernel, router_bias)

    flat_e = eids.reshape(TK)
    perm = jnp.argsort(flat_e, stable=True)
    tok_sorted = perm // K
    g_sorted = flat_e[perm]
    counts = jnp.bincount(flat_e, length=E)
    padded = ((counts + RB - 1) // RB) * RB
    cstarts = jnp.concatenate([jnp.zeros((1,), jnp.int32),
                               jnp.cumsum(padded)[:-1].astype(jnp.int32)])
    ustarts = jnp.concatenate([jnp.zeros((1,), jnp.int32),
                               jnp.cumsum(counts)[:-1].astype(jnp.int32)])
    j = jnp.arange(TK, dtype=jnp.int32)
    pos = cstarts[g_sorted] + j - ustarts[g_sorted]
    rows_padded = jnp.zeros((PAD_ROWS,), jnp.int32).at[pos].set(tok_sorted)
    inv = jnp.zeros((TK,), jnp.int32).at[perm].set(pos).reshape(T, K)
    cends = cstarts + padded
    tile_expert = jnp.minimum(
        jnp.searchsorted(cends, jnp.arange(NT, dtype=jnp.int32) * RB,
                         side="right"),
        E - 1).astype(jnp.int32)

    x_sorted = flat[rows_padded]

    gu4 = gate_up_proj.reshape(E, H, M, 2)
    wg = gu4[..., 0].astype(jnp.bfloat16)
    wu = gu4[..., 1].astype(jnp.bfloat16)
    w2 = down_proj.astype(jnp.bfloat16)
    gub = gate_up_proj_bias.reshape(E, M, 2)
    bg = gub[..., 0].reshape(E, 1, M)
    bu = gub[..., 1].reshape(E, 1, M)
    bd = down_proj_bias.reshape(E, 1, H)

    out_sorted = pl.pallas_call(
        _moe_body,
        grid_spec=pltpu.PrefetchScalarGridSpec(
            num_scalar_prefetch=1,
            grid=(NT,),
            in_specs=[
                pl.BlockSpec((RB, H), lambda i, te: (i, 0)),
                pl.BlockSpec((1, H, M), lambda i, te: (te[i], 0, 0)),
                pl.BlockSpec((1, H, M), lambda i, te: (te[i], 0, 0)),
                pl.BlockSpec((1, M, H), lambda i, te: (te[i], 0, 0)),
                pl.BlockSpec((1, 1, M), lambda i, te: (te[i], 0, 0)),
                pl.BlockSpec((1, 1, M), lambda i, te: (te[i], 0, 0)),
                pl.BlockSpec((1, 1, H), lambda i, te: (te[i], 0, 0)),
            ],
            out_specs=pl.BlockSpec((RB, H), lambda i, te: (i, 0)),
        ),
        out_shape=jax.ShapeDtypeStruct((PAD_ROWS, H), jnp.float32),
    )(tile_expert, x_sorted, wg, wu, w2, bg, bu, bd)

    final = (ws[:, 0:1] * out_sorted[inv[:, 0]]
             + ws[:, 1:2] * out_sorted[inv[:, 1]])
    return final.reshape(B, S, H), scores
